# Initial kernel scaffold; baseline (speedup 1.0000x reference)
import jax
import jax.numpy as jnp
from jax.experimental import pallas as pl


def kernel(node_type, edge_index, edge_type, norm, sketchs, emb_table, bases0, w_comp0, bases1, w_comp1, W_out, b_out):
    raise NotImplementedError("write your pallas kernel here")


# speedup vs baseline: 4.6085x; 4.6085x over previous
import functools

import jax
import jax.numpy as jnp
from jax import lax
from jax.experimental import pallas as pl
from jax.experimental.pallas import tpu as pltpu
from jax.experimental.pallas import tpu_sc as plsc

N = 10000
E = 320000
H = 128
QW = 512

NC = 2
NS = 16
NW = NC * NS
EPW = E // NW
GROUPS = EPW // 16
RPB = 624
RPC = 640

_MESH = plsc.VectorSubcoreMesh(core_axis_name="c", subcore_axis_name="s")


def _hist_body(src_h, dst_h, et_h, norm_h, ntype_h, zeros_h, out_h,
               src_v, dst_v, et_v, norm_v, nt_v, acc):
    c = lax.axis_index("c")
    s = lax.axis_index("s")
    wid = s * NC + c
    base = wid * EPW

    pltpu.sync_copy(src_h.at[pl.ds(base, EPW)], src_v)
    pltpu.sync_copy(dst_h.at[pl.ds(base, EPW)], dst_v)
    pltpu.sync_copy(et_h.at[pl.ds(base, EPW)], et_v)
    pltpu.sync_copy(norm_h.at[pl.ds(base, EPW)], norm_v)
    pltpu.sync_copy(ntype_h, nt_v)

    seg = (N * H) // NS
    pltpu.sync_copy(zeros_h.at[pl.ds(s * seg, seg)], acc.at[pl.ds(s * seg, seg)])
    plsc.subcore_barrier()

    def body(g, carry):
        o = g * 16
        s16 = src_v[pl.ds(o, 16)]
        d16 = dst_v[pl.ds(o, 16)]
        e16 = et_v[pl.ds(o, 16)]
        nt16 = plsc.load_gather(nt_v, [s16])
        flat = d16 * H + e16 * 16 + nt16
        pltpu.sync_copy(norm_v.at[pl.ds(o, 16)], acc.at[flat], add=True)
        return carry

    lax.fori_loop(0, GROUPS, body, 0)
    plsc.subcore_barrier()
    pltpu.sync_copy(acc.at[pl.ds(s * seg, seg)], out_h.at[c, pl.ds(s * seg, seg)])


_SC_PARAMS = pltpu.CompilerParams(needs_layout_passes=False)

_hist = pl.kernel(
    _hist_body,
    out_type=jax.ShapeDtypeStruct((NC, N * H), jnp.float32),
    mesh=_MESH,
    compiler_params=_SC_PARAMS,
    scratch_types=[
        pltpu.VMEM((EPW,), jnp.int32),
        pltpu.VMEM((EPW,), jnp.int32),
        pltpu.VMEM((EPW,), jnp.int32),
        pltpu.VMEM((EPW,), jnp.float32),
        pltpu.VMEM((N,), jnp.int32),
        pltpu.VMEM_SHARED((N * H,), jnp.float32),
    ],
)


def _bag_body(src_h, dst_h, et_h, norm_h, wc1_h, q_h, zeros_h, out_h,
              src_v, dst_v, et_v, norm_v, wc1_v, rows_v, out_v, acc):
    c = lax.axis_index("c")
    s = lax.axis_index("s")
    wid = s * NC + c
    base = wid * EPW

    pltpu.sync_copy(wc1_h, wc1_v)
    pltpu.sync_copy(zeros_h.at[pl.ds(s * RPB, RPC)], acc.at[pl.ds(s * RPB, RPC)])
    plsc.subcore_barrier()

    def chunk(ci, carry):
        cbase = base + ci * ECH
        pltpu.sync_copy(src_h.at[pl.ds(cbase, ECH)], src_v)
        pltpu.sync_copy(dst_h.at[pl.ds(cbase, ECH)], dst_v)
        pltpu.sync_copy(et_h.at[pl.ds(cbase, ECH)], et_v)
        pltpu.sync_copy(norm_h.at[pl.ds(cbase, ECH)], norm_v)

        def body(g, carry2):
            o = g * 16
            s16 = src_v[pl.ds(o, 16)]
            d16 = dst_v[pl.ds(o, 16)]
            e16 = et_v[pl.ds(o, 16)]
            n16 = norm_v[pl.ds(o, 16)]
            pltpu.sync_copy(q_h.at[s16], rows_v)
            cv = [n16 * plsc.load_gather(wc1_v, [e16 + (b * 8)])
                  for b in range(4)]

            def jstep(j, carry3):
                jo = j * 16
                for e in range(16):
                    v = (cv[0][e] * rows_v[e, pl.ds(jo, 16)]
                         + cv[1][e] * rows_v[e, pl.ds(H + jo, 16)]
                         + cv[2][e] * rows_v[e, pl.ds(2 * H + jo, 16)]
                         + cv[3][e] * rows_v[e, pl.ds(3 * H + jo, 16)])
                    out_v[e, pl.ds(jo, 16)] = v
                return carry3

            lax.fori_loop(0, 8, jstep, 0)
            pltpu.sync_copy(out_v, acc.at[d16], add=True)
            return carry2

        lax.fori_loop(0, ECH // 16, body, 0)
        return carry

    lax.fori_loop(0, EPW // ECH, chunk, 0)
    plsc.subcore_barrier()
    pltpu.sync_copy(acc.at[pl.ds(s * RPB, RPC)], out_h.at[c, pl.ds(s * RPB, RPC)])


ECH = 2000

_bag = pl.kernel(
    _bag_body,
    out_type=jax.ShapeDtypeStruct((NC, N, H), jnp.float32),
    mesh=_MESH,
    compiler_params=_SC_PARAMS,
    scratch_types=[
        pltpu.VMEM((ECH,), jnp.int32),
        pltpu.VMEM((ECH,), jnp.int32),
        pltpu.VMEM((ECH,), jnp.int32),
        pltpu.VMEM((ECH,), jnp.float32),
        pltpu.VMEM((32,), jnp.float32),
        pltpu.VMEM((16, QW), jnp.float32),
        pltpu.VMEM((16, H), jnp.float32),
        pltpu.VMEM_SHARED((N, H), jnp.float32),
    ],
)


NBLK = 10
BLK = N // NBLK


def _mid_body(w_ref, emb_ref, b0_ref, wc0_ref, b1_ref, q_ref, m0):
    i = pl.program_id(0)

    @pl.when(i == 0)
    def _():
        for r in range(8):
            w0r = (wc0_ref[r, 0] * b0_ref[0] + wc0_ref[r, 1] * b0_ref[1]
                   + wc0_ref[r, 2] * b0_ref[2] + wc0_ref[r, 3] * b0_ref[3])
            m0[pl.ds(r * 16, 16), :] = jnp.dot(
                emb_ref[...], w0r, preferred_element_type=jnp.float32)

    h = jax.nn.relu(jnp.dot(w_ref[0] + w_ref[1], m0[...],
                            preferred_element_type=jnp.float32))
    for b in range(4):
        q_ref[:, pl.ds(b * H, H)] = jnp.dot(h, b1_ref[b],
                                            preferred_element_type=jnp.float32)


def _mid(wmat2, emb, b0, wc0, b1):
    return pl.pallas_call(
        _mid_body,
        grid=(NBLK,),
        in_specs=[
            pl.BlockSpec((NC, BLK, H), lambda i: (0, i, 0)),
            pl.BlockSpec((16, H), lambda i: (0, 0)),
            pl.BlockSpec((4, H, H), lambda i: (0, 0, 0)),
            pl.BlockSpec(memory_space=pltpu.SMEM),
            pl.BlockSpec((4, H, H), lambda i: (0, 0, 0)),
        ],
        out_specs=pl.BlockSpec((BLK, QW), lambda i: (i, 0)),
        out_shape=jax.ShapeDtypeStruct((N, QW), jnp.float32),
        scratch_shapes=[pltpu.VMEM((H, H), jnp.float32)],
    )(wmat2, emb, b0, wc0, b1)


def _final_body(a_ref, sk_ref, wout_ref, bout_ref, o_ref, mx):
    i = pl.program_id(0)

    @pl.when(i == 0)
    def _():
        mx[...] = jnp.zeros_like(mx)

    h = jax.nn.relu(a_ref[0] + a_ref[1])
    part = jnp.max(h.reshape(BLK // 8, 8, H), axis=0)
    mx[...] = jnp.maximum(mx[...], part)

    @pl.when(i == NBLK - 1)
    def _():
        g = jnp.max(mx[...], axis=0, keepdims=True)
        colsum = jnp.sum(sk_ref[...], axis=0, keepdims=True)
        scores = lax.dot_general(
            g, colsum, (((0,), (0,)), ((), ())),
            preferred_element_type=jnp.float32) * (1.0 / (H ** 0.5))
        m_ = jnp.max(scores, axis=1, keepdims=True)
        ex = jnp.exp(scores - m_)
        p = ex / jnp.sum(ex, axis=1, keepdims=True)
        zcol = lax.dot_general(p, colsum, (((1,), (1,)), ((), ())),
                               preferred_element_type=jnp.float32)
        out = (jnp.dot(g, wout_ref[...], preferred_element_type=jnp.float32)
               + lax.dot_general(zcol, wout_ref[...], (((0,), (0,)), ((), ())),
                                 preferred_element_type=jnp.float32)
               + bout_ref[...])
        o_ref[...] = out


def _final(acc2, sk, wout, bout2d):
    return pl.pallas_call(
        _final_body,
        grid=(NBLK,),
        in_specs=[
            pl.BlockSpec((NC, BLK, H), lambda i: (0, i, 0)),
            pl.BlockSpec((8, H), lambda i: (0, 0)),
            pl.BlockSpec((H, H), lambda i: (0, 0)),
            pl.BlockSpec((1, H), lambda i: (0, 0)),
        ],
        out_specs=pl.BlockSpec((1, H), lambda i: (0, 0)),
        out_shape=jax.ShapeDtypeStruct((1, H), jnp.float32),
        scratch_shapes=[pltpu.VMEM((8, H), jnp.float32)],
    )(acc2, sk, wout, bout2d)


def kernel(node_type, edge_index, edge_type, norm, sketchs, emb_table,
           bases0, w_comp0, bases1, w_comp1, W_out, b_out):
    src = edge_index[0].astype(jnp.int32)
    dst = edge_index[1].astype(jnp.int32)
    et = edge_type.astype(jnp.int32)
    nt = node_type.astype(jnp.int32)
    nrm = norm.reshape(E)

    zeros_flat = jnp.zeros((N * H,), jnp.float32)
    wmat2 = _hist(src, dst, et, nrm, nt, zeros_flat)
    wmat2 = wmat2.reshape(NC, N, H)

    q = _mid(wmat2, emb_table, bases0, w_comp0, bases1)

    wc1t = w_comp1.T.reshape(32)
    zeros2d = jnp.zeros((N, H), jnp.float32)
    acc2 = _bag(src, dst, et, nrm, wc1t, q, zeros2d)

    return _final(acc2, sketchs, W_out, b_out.reshape(1, H))


# speedup vs baseline: 7.8813x; 1.7102x over previous
import functools

import jax
import jax.numpy as jnp
from jax import lax
from jax.experimental import pallas as pl
from jax.experimental.pallas import tpu as pltpu
from jax.experimental.pallas import tpu_sc as plsc

N = 10000
E = 320000
H = 128
QW = 512

NC = 2
NS = 16
NW = NC * NS
EPW = E // NW
GROUPS = EPW // 16
RPB = 624
RPC = 640

_MESH = plsc.VectorSubcoreMesh(core_axis_name="c", subcore_axis_name="s")


def _unpack(p16):
    s16 = jnp.right_shift(p16, 17)
    d16 = jnp.bitwise_and(jnp.right_shift(p16, 3), 16383)
    e16 = jnp.bitwise_and(p16, 7)
    return s16, d16, e16


def _hist_body(pk_h, norm_h, ntype_h, zeros_h, out_h,
               pk_v, norm_v, nt_v, acc):
    c = lax.axis_index("c")
    s = lax.axis_index("s")
    wid = s * NC + c
    base = wid * EPW

    pltpu.sync_copy(pk_h.at[pl.ds(base, EPW)], pk_v)
    pltpu.sync_copy(norm_h.at[pl.ds(base, EPW)], norm_v)
    pltpu.sync_copy(ntype_h, nt_v)

    seg = (N * H) // NS
    pltpu.sync_copy(zeros_h.at[pl.ds(s * seg, seg)], acc.at[pl.ds(s * seg, seg)])
    plsc.subcore_barrier()

    def body(g, carry):
        o = g * 16
        s16, d16, e16 = _unpack(pk_v[pl.ds(o, 16)])
        nt16 = plsc.load_gather(nt_v, [s16])
        flat = d16 * H + e16 * 16 + nt16
        pltpu.sync_copy(norm_v.at[pl.ds(o, 16)], acc.at[flat], add=True)
        return carry

    lax.fori_loop(0, GROUPS, body, 0)
    plsc.subcore_barrier()
    pltpu.sync_copy(acc.at[pl.ds(s * seg, seg)], out_h.at[c, pl.ds(s * seg, seg)])


_SC_PARAMS = pltpu.CompilerParams(needs_layout_passes=False)

_hist = pl.kernel(
    _hist_body,
    out_type=jax.ShapeDtypeStruct((NC, N * H), jnp.float32),
    mesh=_MESH,
    compiler_params=_SC_PARAMS,
    scratch_types=[
        pltpu.VMEM((EPW,), jnp.int32),
        pltpu.VMEM((EPW,), jnp.float32),
        pltpu.VMEM((N,), jnp.int32),
        pltpu.VMEM_SHARED((N * H,), jnp.float32),
    ],
)


def _bag_body(pk_h, norm_h, wc1_h, q_h, zeros_h, out_h,
              pk_v, norm_v, wc1_v, rows0, rows1, out_v,
              sem0, sem1, acc):
    c = lax.axis_index("c")
    s = lax.axis_index("s")
    wid = s * NC + c
    base = wid * EPW

    pltpu.sync_copy(pk_h.at[pl.ds(base, EPW)], pk_v)
    pltpu.sync_copy(norm_h.at[pl.ds(base, EPW)], norm_v)
    pltpu.sync_copy(wc1_h, wc1_v)
    pltpu.sync_copy(zeros_h.at[pl.ds(s * RPB, RPC)], acc.at[pl.ds(s * RPB, RPC)])
    plsc.subcore_barrier()

    def gather(o, rows, sem):
        s16 = jnp.right_shift(pk_v[pl.ds(o, 16)], 17)
        pltpu.async_copy(q_h.at[s16], rows, sem)

    def wait(rows, sem):
        pltpu.make_async_copy(q_h.at[pl.ds(0, 16)], rows, sem).wait()

    def combine_scatter(o, rows):
        _, d16, e16 = _unpack(pk_v[pl.ds(o, 16)])
        n16 = norm_v[pl.ds(o, 16)]
        cv = [n16 * plsc.load_gather(wc1_v, [e16 + (b * 8)])
              for b in range(4)]

        def jstep(j, carry):
            jo = j * 16
            for e in range(16):
                v = (cv[0][e] * rows[e, pl.ds(jo, 16)]
                     + cv[1][e] * rows[e, pl.ds(H + jo, 16)]
                     + cv[2][e] * rows[e, pl.ds(2 * H + jo, 16)]
                     + cv[3][e] * rows[e, pl.ds(3 * H + jo, 16)])
                out_v[e, pl.ds(jo, 16)] = v
            return carry

        lax.fori_loop(0, 8, jstep, 0)
        pltpu.sync_copy(out_v, acc.at[d16], add=True)

    gather(0, rows0, sem0)

    def pair(i, carry):
        o0 = i * 32
        gather(o0 + 16, rows1, sem1)
        wait(rows0, sem0)
        combine_scatter(o0, rows0)
        gather(o0 + 32, rows0, sem0)
        wait(rows1, sem1)
        combine_scatter(o0 + 16, rows1)
        return carry

    lax.fori_loop(0, GROUPS // 2, pair, 0)
    wait(rows0, sem0)
    combine_scatter((GROUPS - 1) * 16, rows0)

    plsc.subcore_barrier()
    pltpu.sync_copy(acc.at[pl.ds(s * RPB, RPC)], out_h.at[c, pl.ds(s * RPB, RPC)])


_bag = pl.kernel(
    _bag_body,
    out_type=jax.ShapeDtypeStruct((NC, N, H), jnp.float32),
    mesh=_MESH,
    compiler_params=_SC_PARAMS,
    scratch_types=[
        pltpu.VMEM((EPW,), jnp.int32),
        pltpu.VMEM((EPW,), jnp.float32),
        pltpu.VMEM((32,), jnp.float32),
        pltpu.VMEM((16, QW), jnp.float32),
        pltpu.VMEM((16, QW), jnp.float32),
        pltpu.VMEM((16, H), jnp.float32),
        pltpu.SemaphoreType.DMA,
        pltpu.SemaphoreType.DMA,
        pltpu.VMEM_SHARED((N, H), jnp.float32),
    ],
)


NBLK = 10
BLK = N // NBLK


def _mid_body(w_ref, emb_ref, b0_ref, wc0_ref, b1_ref, q_ref, m0):
    i = pl.program_id(0)

    @pl.when(i == 0)
    def _():
        for r in range(8):
            w0r = (wc0_ref[r, 0] * b0_ref[0] + wc0_ref[r, 1] * b0_ref[1]
                   + wc0_ref[r, 2] * b0_ref[2] + wc0_ref[r, 3] * b0_ref[3])
            m0[pl.ds(r * 16, 16), :] = jnp.dot(
                emb_ref[...], w0r, preferred_element_type=jnp.float32)

    h = jax.nn.relu(jnp.dot(w_ref[0] + w_ref[1], m0[...],
                            preferred_element_type=jnp.float32))
    for b in range(4):
        q_ref[:, pl.ds(b * H, H)] = jnp.dot(h, b1_ref[b],
                                            preferred_element_type=jnp.float32)


def _mid(wmat2, emb, b0, wc0, b1):
    return pl.pallas_call(
        _mid_body,
        grid=(NBLK,),
        in_specs=[
            pl.BlockSpec((NC, BLK, H), lambda i: (0, i, 0)),
            pl.BlockSpec((16, H), lambda i: (0, 0)),
            pl.BlockSpec((4, H, H), lambda i: (0, 0, 0)),
            pl.BlockSpec(memory_space=pltpu.SMEM),
            pl.BlockSpec((4, H, H), lambda i: (0, 0, 0)),
        ],
        out_specs=pl.BlockSpec((BLK, QW), lambda i: (i, 0)),
        out_shape=jax.ShapeDtypeStruct((N, QW), jnp.float32),
        scratch_shapes=[pltpu.VMEM((H, H), jnp.float32)],
    )(wmat2, emb, b0, wc0, b1)


def _final_body(a_ref, sk_ref, wout_ref, bout_ref, o_ref, mx):
    i = pl.program_id(0)

    @pl.when(i == 0)
    def _():
        mx[...] = jnp.zeros_like(mx)

    h = jax.nn.relu(a_ref[0] + a_ref[1])
    part = jnp.max(h.reshape(BLK // 8, 8, H), axis=0)
    mx[...] = jnp.maximum(mx[...], part)

    @pl.when(i == NBLK - 1)
    def _():
        g = jnp.max(mx[...], axis=0, keepdims=True)
        colsum = jnp.sum(sk_ref[...], axis=0, keepdims=True)
        scores = lax.dot_general(
            g, colsum, (((0,), (0,)), ((), ())),
            preferred_element_type=jnp.float32) * (1.0 / (H ** 0.5))
        m_ = jnp.max(scores, axis=1, keepdims=True)
        ex = jnp.exp(scores - m_)
        p = ex / jnp.sum(ex, axis=1, keepdims=True)
        zcol = lax.dot_general(p, colsum, (((1,), (1,)), ((), ())),
                               preferred_element_type=jnp.float32)
        out = (jnp.dot(g, wout_ref[...], preferred_element_type=jnp.float32)
               + lax.dot_general(zcol, wout_ref[...], (((0,), (0,)), ((), ())),
                                 preferred_element_type=jnp.float32)
               + bout_ref[...])
        o_ref[...] = out


def _final(acc2, sk, wout, bout2d):
    return pl.pallas_call(
        _final_body,
        grid=(NBLK,),
        in_specs=[
            pl.BlockSpec((NC, BLK, H), lambda i: (0, i, 0)),
            pl.BlockSpec((8, H), lambda i: (0, 0)),
            pl.BlockSpec((H, H), lambda i: (0, 0)),
            pl.BlockSpec((1, H), lambda i: (0, 0)),
        ],
        out_specs=pl.BlockSpec((1, H), lambda i: (0, 0)),
        out_shape=jax.ShapeDtypeStruct((1, H), jnp.float32),
        scratch_shapes=[pltpu.VMEM((8, H), jnp.float32)],
    )(acc2, sk, wout, bout2d)


def kernel(node_type, edge_index, edge_type, norm, sketchs, emb_table,
           bases0, w_comp0, bases1, w_comp1, W_out, b_out):
    src = edge_index[0].astype(jnp.int32)
    dst = edge_index[1].astype(jnp.int32)
    et = edge_type.astype(jnp.int32)
    nt = node_type.astype(jnp.int32)
    nrm = norm.reshape(E)
    packed = jnp.bitwise_or(
        jnp.left_shift(src, 17),
        jnp.bitwise_or(jnp.left_shift(dst, 3), et))

    zeros_flat = jnp.zeros((N * H,), jnp.float32)
    wmat2 = _hist(packed, nrm, nt, zeros_flat)
    wmat2 = wmat2.reshape(NC, N, H)

    q = _mid(wmat2, emb_table, bases0, w_comp0, bases1)

    wc1t = w_comp1.T.reshape(32)
    zeros2d = jnp.zeros((N, H), jnp.float32)
    acc2 = _bag(packed, nrm, wc1t, q, zeros2d)

    return _final(acc2, sketchs, W_out, b_out.reshape(1, H))


# speedup vs baseline: 12.0762x; 1.5322x over previous
import functools

import jax
import jax.numpy as jnp
from jax import lax
from jax.experimental import pallas as pl
from jax.experimental.pallas import tpu as pltpu
from jax.experimental.pallas import tpu_sc as plsc

N = 10000
E = 320000
H = 128
ZW = 1024

NC = 2
NS = 16
NW = NC * NS
EPW = E // NW
GROUPS = EPW // 16
RPB = 624
RPC = 640

_MESH = plsc.VectorSubcoreMesh(core_axis_name="c", subcore_axis_name="s")


def _unpack(p16):
    s16 = jnp.right_shift(p16, 17)
    d16 = jnp.bitwise_and(jnp.right_shift(p16, 3), 16383)
    e16 = jnp.bitwise_and(p16, 7)
    return s16, d16, e16


def _hist_body(pk_h, norm_h, ntype_h, zeros_h, out_h,
               pk_v, norm_v, nt_v, fidx_v, acc):
    c = lax.axis_index("c")
    s = lax.axis_index("s")
    wid = s * NC + c
    base = wid * EPW

    pltpu.sync_copy(pk_h.at[pl.ds(base, EPW)], pk_v)
    pltpu.sync_copy(norm_h.at[pl.ds(base, EPW)], norm_v)
    pltpu.sync_copy(ntype_h, nt_v)

    seg = (N * H) // NS
    pltpu.sync_copy(zeros_h.at[pl.ds(s * seg, seg)], acc.at[pl.ds(s * seg, seg)])
    plsc.subcore_barrier()

    def body(g, carry):
        o = g * 128
        for k in range(8):
            s16, d16, e16 = _unpack(pk_v[pl.ds(o + k * 16, 16)])
            nt16 = plsc.load_gather(nt_v, [s16])
            fidx_v[pl.ds(k * 16, 16)] = d16 * H + e16 * 16 + nt16
        pltpu.sync_copy(norm_v.at[pl.ds(o, 128)], acc.at[fidx_v], add=True)
        return carry

    lax.fori_loop(0, EPW // 128, body, 0)
    o = (EPW // 128) * 128
    s16, d16, e16 = _unpack(pk_v[pl.ds(o, 16)])
    nt16 = plsc.load_gather(nt_v, [s16])
    flat = d16 * H + e16 * 16 + nt16
    pltpu.sync_copy(norm_v.at[pl.ds(o, 16)], acc.at[flat], add=True)
    plsc.subcore_barrier()
    pltpu.sync_copy(acc.at[pl.ds(s * seg, seg)], out_h.at[c, pl.ds(s * seg, seg)])


_SC_PARAMS = pltpu.CompilerParams(needs_layout_passes=False)

_hist = pl.kernel(
    _hist_body,
    out_type=jax.ShapeDtypeStruct((NC, N * H), jnp.float32),
    mesh=_MESH,
    compiler_params=_SC_PARAMS,
    scratch_types=[
        pltpu.VMEM((EPW,), jnp.int32),
        pltpu.VMEM((EPW,), jnp.float32),
        pltpu.VMEM((N,), jnp.int32),
        pltpu.VMEM((128,), jnp.int32),
        pltpu.VMEM_SHARED((N * H,), jnp.float32),
    ],
)


def _bag_body(pk_h, norm_h, z_h, zeros_h, out_h,
              pk_v, norm_v, rows0, rows1, out0, out1,
              sem0, sem1, acc):
    c = lax.axis_index("c")
    s = lax.axis_index("s")
    wid = s * NC + c
    base = wid * EPW

    pltpu.sync_copy(pk_h.at[pl.ds(base, EPW)], pk_v)
    pltpu.sync_copy(norm_h.at[pl.ds(base, EPW)], norm_v)
    pltpu.sync_copy(zeros_h.at[pl.ds(s * RPB, RPC)], acc.at[pl.ds(s * RPB, RPC)])
    plsc.subcore_barrier()

    def gather(o, rows, sem):
        p16 = pk_v[pl.ds(o, 16)]
        idx = jnp.right_shift(p16, 17) * 8 + jnp.bitwise_and(p16, 7)
        pltpu.async_copy(z_h.at[idx], rows, sem)

    def wait(rows, sem):
        pltpu.make_async_copy(z_h.at[pl.ds(0, 16)], rows, sem).wait()

    def combine_scatter(o, rows, out_v):
        d16 = jnp.bitwise_and(jnp.right_shift(pk_v[pl.ds(o, 16)], 3), 16383)
        n16 = norm_v[pl.ds(o, 16)]

        def jstep(j, carry):
            jo = j * 16
            for e in range(16):
                out_v[e, pl.ds(jo, 16)] = n16[e] * rows[e, pl.ds(jo, 16)]
            return carry

        lax.fori_loop(0, 8, jstep, 0)
        pltpu.sync_copy(out_v, acc.at[d16], add=True)

    gather(0, rows0, sem0)

    def pair(i, carry):
        o0 = i * 32
        gather(o0 + 16, rows1, sem1)
        wait(rows0, sem0)
        combine_scatter(o0, rows0, out0)
        gather(o0 + 32, rows0, sem0)
        wait(rows1, sem1)
        combine_scatter(o0 + 16, rows1, out1)
        return carry

    lax.fori_loop(0, GROUPS // 2, pair, 0)
    wait(rows0, sem0)
    combine_scatter((GROUPS - 1) * 16, rows0, out0)

    plsc.subcore_barrier()
    pltpu.sync_copy(acc.at[pl.ds(s * RPB, RPC)], out_h.at[c, pl.ds(s * RPB, RPC)])


_bag = pl.kernel(
    _bag_body,
    out_type=jax.ShapeDtypeStruct((NC, N, H), jnp.float32),
    mesh=_MESH,
    compiler_params=_SC_PARAMS,
    scratch_types=[
        pltpu.VMEM((EPW,), jnp.int32),
        pltpu.VMEM((EPW,), jnp.float32),
        pltpu.VMEM((16, H), jnp.float32),
        pltpu.VMEM((16, H), jnp.float32),
        pltpu.VMEM((16, H), jnp.float32),
        pltpu.VMEM((16, H), jnp.float32),
        pltpu.SemaphoreType.DMA,
        pltpu.SemaphoreType.DMA,
        pltpu.VMEM_SHARED((N, H), jnp.float32),
    ],
)


NBLK = 10
BLK = N // NBLK


def _mid_body(w_ref, emb_ref, b0_ref, wc0_ref, b1_ref, wc1_ref, q_ref, m0, w1s):
    i = pl.program_id(0)

    @pl.when(i == 0)
    def _():
        for r in range(8):
            w0r = (wc0_ref[r, 0] * b0_ref[0] + wc0_ref[r, 1] * b0_ref[1]
                   + wc0_ref[r, 2] * b0_ref[2] + wc0_ref[r, 3] * b0_ref[3])
            m0[pl.ds(r * 16, 16), :] = jnp.dot(
                emb_ref[...], w0r, preferred_element_type=jnp.float32)
            w1s[r] = (wc1_ref[r, 0] * b1_ref[0] + wc1_ref[r, 1] * b1_ref[1]
                      + wc1_ref[r, 2] * b1_ref[2] + wc1_ref[r, 3] * b1_ref[3])

    h = jax.nn.relu(jnp.dot(w_ref[0] + w_ref[1], m0[...],
                            preferred_element_type=jnp.float32))
    for r in range(8):
        q_ref[:, pl.ds(r * H, H)] = jnp.dot(h, w1s[r],
                                            preferred_element_type=jnp.float32)


def _mid(wmat2, emb, b0, wc0, b1, wc1):
    return pl.pallas_call(
        _mid_body,
        grid=(NBLK,),
        in_specs=[
            pl.BlockSpec((NC, BLK, H), lambda i: (0, i, 0)),
            pl.BlockSpec((16, H), lambda i: (0, 0)),
            pl.BlockSpec((4, H, H), lambda i: (0, 0, 0)),
            pl.BlockSpec(memory_space=pltpu.SMEM),
            pl.BlockSpec((4, H, H), lambda i: (0, 0, 0)),
            pl.BlockSpec(memory_space=pltpu.SMEM),
        ],
        out_specs=pl.BlockSpec((BLK, ZW), lambda i: (i, 0)),
        out_shape=jax.ShapeDtypeStruct((N, ZW), jnp.float32),
        scratch_shapes=[pltpu.VMEM((H, H), jnp.float32),
                        pltpu.VMEM((8, H, H), jnp.float32)],
    )(wmat2, emb, b0, wc0, b1, wc1)


def _final_body(a_ref, sk_ref, wout_ref, bout_ref, o_ref, mx):
    i = pl.program_id(0)

    @pl.when(i == 0)
    def _():
        mx[...] = jnp.zeros_like(mx)

    h = jax.nn.relu(a_ref[0] + a_ref[1])
    part = jnp.max(h.reshape(BLK // 8, 8, H), axis=0)
    mx[...] = jnp.maximum(mx[...], part)

    @pl.when(i == NBLK - 1)
    def _():
        g = jnp.max(mx[...], axis=0, keepdims=True)
        colsum = jnp.sum(sk_ref[...], axis=0, keepdims=True)
        scores = lax.dot_general(
            g, colsum, (((0,), (0,)), ((), ())),
            preferred_element_type=jnp.float32) * (1.0 / (H ** 0.5))
        m_ = jnp.max(scores, axis=1, keepdims=True)
        ex = jnp.exp(scores - m_)
        p = ex / jnp.sum(ex, axis=1, keepdims=True)
        zcol = lax.dot_general(p, colsum, (((1,), (1,)), ((), ())),
                               preferred_element_type=jnp.float32)
        out = (jnp.dot(g, wout_ref[...], preferred_element_type=jnp.float32)
               + lax.dot_general(zcol, wout_ref[...], (((0,), (0,)), ((), ())),
                                 preferred_element_type=jnp.float32)
               + bout_ref[...])
        o_ref[...] = out


def _final(acc2, sk, wout, bout2d):
    return pl.pallas_call(
        _final_body,
        grid=(NBLK,),
        in_specs=[
            pl.BlockSpec((NC, BLK, H), lambda i: (0, i, 0)),
            pl.BlockSpec((8, H), lambda i: (0, 0)),
            pl.BlockSpec((H, H), lambda i: (0, 0)),
            pl.BlockSpec((1, H), lambda i: (0, 0)),
        ],
        out_specs=pl.BlockSpec((1, H), lambda i: (0, 0)),
        out_shape=jax.ShapeDtypeStruct((1, H), jnp.float32),
        scratch_shapes=[pltpu.VMEM((8, H), jnp.float32)],
    )(acc2, sk, wout, bout2d)


def kernel(node_type, edge_index, edge_type, norm, sketchs, emb_table,
           bases0, w_comp0, bases1, w_comp1, W_out, b_out):
    src = edge_index[0].astype(jnp.int32)
    dst = edge_index[1].astype(jnp.int32)
    et = edge_type.astype(jnp.int32)
    nt = node_type.astype(jnp.int32)
    nrm = norm.reshape(E)
    packed = jnp.bitwise_or(
        jnp.left_shift(src, 17),
        jnp.bitwise_or(jnp.left_shift(dst, 3), et))

    zeros_flat = jnp.zeros((N * H,), jnp.float32)
    wmat2 = _hist(packed, nrm, nt, zeros_flat)
    wmat2 = wmat2.reshape(NC, N, H)

    z = _mid(wmat2, emb_table, bases0, w_comp0, bases1, w_comp1)
    z = z.reshape(N * 8, H)

    zeros2d = jnp.zeros((N, H), jnp.float32)
    acc2 = _bag(packed, nrm, z, zeros2d)

    return _final(acc2, sketchs, W_out, b_out.reshape(1, H))
